# baseline (speedup 1.0000x reference)
import functools

import jax
import jax.numpy as jnp
from jax import lax
from jax.experimental import pallas as pl
from jax.experimental.pallas import tpu as pltpu
from jax.experimental.pallas import tpu_sc as plsc

D0, D1, D2 = 16, 32, 64
BATCH = 16384

_info = plsc.get_sparse_core_info()
_NC, _NS = _info.num_cores, _info.num_subcores
_NW = _NC * _NS
_BPW = BATCH // _NW

_mesh = plsc.VectorSubcoreMesh(core_axis_name="c", subcore_axis_name="s")


@functools.partial(
    pl.kernel,
    mesh=_mesh,
    out_type=(
        jax.ShapeDtypeStruct((BATCH, D0), jnp.float32),
        jax.ShapeDtypeStruct((BATCH, D1), jnp.float32),
        jax.ShapeDtypeStruct((BATCH, D2), jnp.float32),
    ),
    scratch_types=[
        pltpu.VMEM((_BPW,), jnp.int32),
        pltpu.VMEM((_BPW, D0), jnp.float32),
        pltpu.VMEM((_BPW, D1), jnp.float32),
        pltpu.VMEM((_BPW, D2), jnp.float32),
        pltpu.SemaphoreType.DMA,
    ],
    compiler_params=pltpu.CompilerParams(use_tc_tiling_on_sc=False),
)
def _emb_lookup(idx_hbm, t0_hbm, t1_hbm, t2_hbm, o0_hbm, o1_hbm, o2_hbm,
                idx_v, r0_v, r1_v, r2_v, sem):
    wid = lax.axis_index("s") * _NC + lax.axis_index("c")
    base = wid * _BPW
    pltpu.sync_copy(idx_hbm.at[pl.ds(base, _BPW)], idx_v)
    h0 = pltpu.async_copy(t0_hbm.at[idx_v], r0_v, sem)
    h1 = pltpu.async_copy(t1_hbm.at[idx_v], r1_v, sem)
    h2 = pltpu.async_copy(t2_hbm.at[idx_v], r2_v, sem)
    h0.wait()
    pltpu.sync_copy(r0_v, o0_hbm.at[pl.ds(base, _BPW)])
    h1.wait()
    pltpu.sync_copy(r1_v, o1_hbm.at[pl.ds(base, _BPW)])
    h2.wait()
    pltpu.sync_copy(r2_v, o2_hbm.at[pl.ds(base, _BPW)])


def kernel(task_id, table0, table1, table2):
    return _emb_lookup(task_id.astype(jnp.int32), table0, table1, table2)


# speedup vs baseline: 3.9766x; 3.9766x over previous
import functools

import jax
import jax.numpy as jnp
from jax import lax
from jax.experimental import pallas as pl
from jax.experimental.pallas import tpu as pltpu
from jax.experimental.pallas import tpu_sc as plsc

D0, D1, D2 = 16, 32, 64
NUM_ROWS = 1000000
BATCH = 16384
LANES = 16

_info = plsc.get_sparse_core_info()
_NC, _NS = _info.num_cores, _info.num_subcores
_NW = _NC * _NS
_BPW = BATCH // _NW
_NBUF = 8
_GRP = 128

_mesh = plsc.VectorSubcoreMesh(core_axis_name="c", subcore_axis_name="s")


@functools.partial(
    pl.kernel,
    mesh=_mesh,
    out_type=(
        jax.ShapeDtypeStruct((D0, BATCH), jnp.float32),
        jax.ShapeDtypeStruct((D1, BATCH), jnp.float32),
        jax.ShapeDtypeStruct((D2, BATCH), jnp.float32),
    ),
    scratch_types=[
        pltpu.VMEM((_BPW,), jnp.int32),
        pltpu.VMEM((_NBUF, D0, 128), jnp.float32),
        pltpu.VMEM((_NBUF, D1, 128), jnp.float32),
        pltpu.VMEM((_NBUF, D2, 128), jnp.float32),
        pltpu.VMEM((D0, _GRP), jnp.float32),
        pltpu.VMEM((D1, _GRP), jnp.float32),
        pltpu.VMEM((D2, _GRP), jnp.float32),
        [pltpu.SemaphoreType.DMA] * _NBUF,
    ],
    compiler_params=pltpu.CompilerParams(needs_layout_passes=False),
)
def _emb_lookup(idx_hbm, t0_hbm, t1_hbm, t2_hbm, o0_hbm, o1_hbm, o2_hbm,
                idx_v, rb0, rb1, rb2, s0, s1, s2, sems):
    wid = lax.axis_index("s") * _NC + lax.axis_index("c")
    base = wid * _BPW
    pltpu.sync_copy(idx_hbm.at[pl.ds(base, _BPW)], idx_v)
    iota = lax.iota(jnp.int32, LANES)

    def fetch(slot, col_off):
        off = pl.multiple_of(col_off, 128)
        pltpu.async_copy(t0_hbm.at[:, pl.ds(off, 128)], rb0.at[slot], sems[slot])
        pltpu.async_copy(t1_hbm.at[:, pl.ds(off, 128)], rb1.at[slot], sems[slot])
        pltpu.async_copy(t2_hbm.at[:, pl.ds(off, 128)], rb2.at[slot], sems[slot])

    def wait(slot):
        pltpu.make_async_copy(t0_hbm.at[:, pl.ds(0, 128)], rb0.at[slot], sems[slot]).wait()
        pltpu.make_async_copy(t1_hbm.at[:, pl.ds(0, 128)], rb1.at[slot], sems[slot]).wait()
        pltpu.make_async_copy(t2_hbm.at[:, pl.ds(0, 128)], rb2.at[slot], sems[slot]).wait()

    def extract(slot, lane, kcol):
        lanev = jnp.full((LANES,), lane, jnp.int32)
        kv = jnp.full((LANES,), kcol, jnp.int32)
        v = plsc.load_gather(rb0.at[slot], [iota, lanev])
        plsc.store_scatter(s0, [iota, kv], v)
        for h in range(D1 // LANES):
            v = plsc.load_gather(rb1.at[slot], [iota + h * LANES, lanev])
            plsc.store_scatter(s1, [iota + h * LANES, kv], v)
        for h in range(D2 // LANES):
            v = plsc.load_gather(rb2.at[slot], [iota + h * LANES, lanev])
            plsc.store_scatter(s2, [iota + h * LANES, kv], v)

    def flush(grp_off):
        off = pl.multiple_of(base + grp_off, 128)
        pltpu.sync_copy(s0, o0_hbm.at[:, pl.ds(off, _GRP)])
        pltpu.sync_copy(s1, o1_hbm.at[:, pl.ds(off, _GRP)])
        pltpu.sync_copy(s2, o2_hbm.at[:, pl.ds(off, _GRP)])

    def body(blk, carry):
        cv_prev, lv_prev = carry
        kk0 = blk * LANES
        iv = idx_v[pl.ds(kk0, LANES)]
        lv = jnp.bitwise_and(iv, 127)
        cv = iv - lv
        for j in range(LANES):
            kk = kk0 + j
            if j < _NBUF:
                @pl.when(blk > 0)
                def _():
                    wait(j)
                    extract(j, lv_prev[j + LANES - _NBUF],
                            (kk - _NBUF) % _GRP)
                if j == _NBUF - 1:
                    @pl.when(jnp.logical_and(blk > 0, blk % 8 == 0))
                    def _():
                        flush((blk - 8) * LANES)
                fetch(j, cv[j])
            else:
                wait(j % _NBUF)
                extract(j % _NBUF, lv[j - _NBUF], (kk - _NBUF) % _GRP)
                fetch(j % _NBUF, cv[j])
        return cv, lv

    zero = jnp.zeros((LANES,), jnp.int32)
    cv_last, lv_last = lax.fori_loop(0, _BPW // LANES, body, (zero, zero))

    for j in range(_NBUF):
        kk = _BPW - _NBUF + j
        wait(j % _NBUF)
        extract(j % _NBUF, lv_last[j + LANES - _NBUF], kk % _GRP)
    flush(_BPW - _GRP)


def kernel(task_id, table0, table1, table2):
    o0t, o1t, o2t = _emb_lookup(
        task_id.astype(jnp.int32), table0.T, table1.T, table2.T
    )
    return o0t.T, o1t.T, o2t.T
